# baseline (speedup 1.0000x reference)
import functools

import jax
import jax.numpy as jnp
from jax import lax
from jax.experimental import pallas as pl
from jax.experimental.pallas import tpu as pltpu
from jax.experimental.pallas import tpu_sc as plsc

_SEQ = 200
_BATCH = 4096
_DIM = 64
_NCORES = 2
_NSUB = 16
_NW = _NCORES * _NSUB
_BPW = _BATCH // _NW
_CH0 = 120
_CH1 = _SEQ - _CH0


def _cbow_pool(texts_t, emb):
    mesh = plsc.VectorSubcoreMesh(core_axis_name="c", subcore_axis_name="s")

    @functools.partial(
        pl.kernel,
        out_type=jax.ShapeDtypeStruct((_BATCH, _DIM), jnp.float32),
        mesh=mesh,
        scratch_types=[
            pltpu.VMEM((_BPW, _SEQ), jnp.int32),
            pltpu.VMEM((_SEQ, _DIM), jnp.float32),
            pltpu.VMEM((_BPW, _DIM), jnp.float32),
            pltpu.SemaphoreType.DMA,
        ],
        compiler_params=pltpu.CompilerParams(use_tc_tiling_on_sc=False),
    )
    def kern(texts_hbm, emb_hbm, out_hbm, idx_v, buf, out_v, sem):
        wid = lax.axis_index("c") * _NSUB + lax.axis_index("s")
        base = wid * _BPW
        pltpu.sync_copy(texts_hbm.at[pl.ds(base, _BPW)], idx_v)

        @pl.loop(0, _BPW)
        def _(b):
            c0 = pltpu.async_copy(
                emb_hbm.at[idx_v.at[b, pl.ds(0, _CH0)]],
                buf.at[pl.ds(0, _CH0)], sem)
            c1 = pltpu.async_copy(
                emb_hbm.at[idx_v.at[b, pl.ds(_CH0, _CH1)]],
                buf.at[pl.ds(_CH0, _CH1)], sem)
            c0.wait()
            c1.wait()

            def body(i, acc):
                s = i * 2
                return (
                    acc[0] + buf[s, pl.ds(0, 16)],
                    acc[1] + buf[s, pl.ds(16, 16)],
                    acc[2] + buf[s, pl.ds(32, 16)],
                    acc[3] + buf[s, pl.ds(48, 16)],
                    acc[4] + buf[s + 1, pl.ds(0, 16)],
                    acc[5] + buf[s + 1, pl.ds(16, 16)],
                    acc[6] + buf[s + 1, pl.ds(32, 16)],
                    acc[7] + buf[s + 1, pl.ds(48, 16)],
                )

            z = jnp.zeros((16,), jnp.float32)
            a = lax.fori_loop(0, _SEQ // 2, body, (z,) * 8)
            inv = jnp.float32(1.0 / _SEQ)
            for c in range(4):
                out_v[b, pl.ds(16 * c, 16)] = (a[c] + a[c + 4]) * inv

        pltpu.sync_copy(out_v, out_hbm.at[pl.ds(base, _BPW)])

    return kern(texts_t, emb)


def _mlp_head(cbow, W1, b1, W2, b2):
    def body(x_ref, w1_ref, b1_ref, w2_ref, b2_ref, o_ref):
        x = x_ref[...]
        h = jnp.maximum(
            jnp.dot(x, w1_ref[...], preferred_element_type=jnp.float32)
            + b1_ref[...], 0.0)
        o_ref[...] = (
            jnp.dot(h, w2_ref[...], preferred_element_type=jnp.float32)
            + b2_ref[...])

    return pl.pallas_call(
        body,
        out_shape=jax.ShapeDtypeStruct((_BATCH, b2.shape[-1]), jnp.float32),
    )(cbow, W1, b1.reshape(1, -1), W2, b2.reshape(1, -1))


def kernel(texts, emb, W1, b1, W2, b2):
    texts_t = texts.T.astype(jnp.int32)
    cbow = _cbow_pool(texts_t, emb)
    return _mlp_head(cbow, W1, b1, W2, b2)


# speedup vs baseline: 1.1318x; 1.1318x over previous
import functools

import jax
import jax.numpy as jnp
from jax import lax
from jax.experimental import pallas as pl
from jax.experimental.pallas import tpu as pltpu
from jax.experimental.pallas import tpu_sc as plsc

_SEQ = 200
_BATCH = 4096
_DIM = 64
_NCORES = 2
_NSUB = 16
_NW = _NCORES * _NSUB
_BPW = _BATCH // _NW
_CH0 = 120
_CH1 = _SEQ - _CH0


def _cbow_pool(texts_t, emb):
    mesh = plsc.VectorSubcoreMesh(core_axis_name="c", subcore_axis_name="s")

    @functools.partial(
        pl.kernel,
        out_type=jax.ShapeDtypeStruct((_BATCH, _DIM), jnp.float32),
        mesh=mesh,
        scratch_types=[
            pltpu.VMEM((_BPW, _SEQ), jnp.int32),
            pltpu.VMEM((_SEQ, _DIM), jnp.float32),
            pltpu.VMEM((_SEQ, _DIM), jnp.float32),
            pltpu.VMEM((_BPW, _DIM), jnp.float32),
            pltpu.SemaphoreType.DMA,
            pltpu.SemaphoreType.DMA,
        ],
        compiler_params=pltpu.CompilerParams(use_tc_tiling_on_sc=False),
    )
    def kern(texts_hbm, emb_hbm, out_hbm, idx_v, bufa, bufb, out_v,
             sema, semb):
        wid = lax.axis_index("c") * _NSUB + lax.axis_index("s")
        base = wid * _BPW
        pltpu.sync_copy(texts_hbm.at[pl.ds(base, _BPW)], idx_v)

        def copies(b, buf, sem):
            return (
                pltpu.make_async_copy(
                    emb_hbm.at[idx_v.at[b, pl.ds(0, _CH0)]],
                    buf.at[pl.ds(0, _CH0)], sem),
                pltpu.make_async_copy(
                    emb_hbm.at[idx_v.at[b, pl.ds(_CH0, _CH1)]],
                    buf.at[pl.ds(_CH0, _CH1)], sem),
            )

        def fire(b, buf, sem):
            for c in copies(b, buf, sem):
                c.start()

        def drain(b, buf, sem):
            for c in copies(b, buf, sem):
                c.wait()

        def accum(b, buf):
            def body(i, acc):
                s = i * 2
                return (
                    acc[0] + buf[s, pl.ds(0, 16)],
                    acc[1] + buf[s, pl.ds(16, 16)],
                    acc[2] + buf[s, pl.ds(32, 16)],
                    acc[3] + buf[s, pl.ds(48, 16)],
                    acc[4] + buf[s + 1, pl.ds(0, 16)],
                    acc[5] + buf[s + 1, pl.ds(16, 16)],
                    acc[6] + buf[s + 1, pl.ds(32, 16)],
                    acc[7] + buf[s + 1, pl.ds(48, 16)],
                )

            z = jnp.zeros((16,), jnp.float32)
            a = lax.fori_loop(0, _SEQ // 2, body, (z,) * 8, unroll=2)
            inv = jnp.float32(1.0 / _SEQ)
            for c in range(4):
                out_v[b, pl.ds(16 * c, 16)] = (a[c] + a[c + 4]) * inv

        fire(0, bufa, sema)

        @pl.loop(0, _BPW, step=2)
        def _(b):
            fire(b + 1, bufb, semb)
            drain(b, bufa, sema)
            accum(b, bufa)

            @pl.when(b + 2 < _BPW)
            def _():
                fire(b + 2, bufa, sema)

            drain(b + 1, bufb, semb)
            accum(b + 1, bufb)

        pltpu.sync_copy(out_v, out_hbm.at[pl.ds(base, _BPW)])

    return kern(texts_t, emb)


def _mlp_head(cbow, W1, b1, W2, b2):
    def body(x_ref, w1_ref, b1_ref, w2_ref, b2_ref, o_ref):
        x = x_ref[...]
        h = jnp.maximum(
            jnp.dot(x, w1_ref[...], preferred_element_type=jnp.float32)
            + b1_ref[...], 0.0)
        o_ref[...] = (
            jnp.dot(h, w2_ref[...], preferred_element_type=jnp.float32)
            + b2_ref[...])

    return pl.pallas_call(
        body,
        out_shape=jax.ShapeDtypeStruct((_BATCH, b2.shape[-1]), jnp.float32),
    )(cbow, W1, b1.reshape(1, -1), W2, b2.reshape(1, -1))


def kernel(texts, emb, W1, b1, W2, b2):
    texts_t = texts.T.astype(jnp.int32)
    cbow = _cbow_pool(texts_t, emb)
    return _mlp_head(cbow, W1, b1, W2, b2)
